# Initial kernel scaffold; baseline (speedup 1.0000x reference)
import jax
import jax.numpy as jnp
from jax.experimental import pallas as pl


def kernel(target_bbox, rpn_match, rpn_bbox):
    raise NotImplementedError("write your pallas kernel here")


# speedup vs baseline: 40.1715x; 40.1715x over previous
import functools

import jax
import jax.numpy as jnp
from jax import lax
from jax.experimental import pallas as pl
from jax.experimental.pallas import tpu as pltpu
from jax.experimental.pallas import tpu_sc as plsc

_NC = 2
_NS = 16
_NW = _NC * _NS
_L = 16


def _sc_partials(match_flat, tgt_flat, bbox_flat, *, A, M):
    chunk_len = A // 4
    steps = chunk_len // _L

    mesh = plsc.VectorSubcoreMesh(core_axis_name="c", subcore_axis_name="s")

    @functools.partial(
        pl.kernel,
        mesh=mesh,
        compiler_params=pltpu.CompilerParams(needs_layout_passes=False),
        out_type=jax.ShapeDtypeStruct((2 * _NW, _L), jnp.float32),
        scratch_types=[
            pltpu.VMEM((chunk_len,), jnp.int32),
            pltpu.VMEM((4 * chunk_len,), jnp.float32),
            pltpu.VMEM((4 * M,), jnp.float32),
            pltpu.VMEM((_L,), jnp.int32),
            pltpu.VMEM((_L,), jnp.float32),
            pltpu.VMEM_SHARED((_NS, _L), jnp.int32),
            pltpu.SemaphoreType.DMA,
        ],
    )
    def body(match_hbm, tgt_hbm, bbox_hbm, out_hbm,
             mchunk, bchunk, trow, stage_i, stage_f, shared_cnt, sem):
        c = lax.axis_index("c")
        s = lax.axis_index("s")
        wid = c * _NS + s
        row = wid // 4
        chunk = s % 4
        iota = lax.iota(jnp.int32, _L)

        bcopy = pltpu.make_async_copy(
            bbox_hbm.at[pl.ds(wid * 4 * chunk_len, 4 * chunk_len)], bchunk,
            sem)
        bcopy.start()
        pltpu.sync_copy(match_hbm.at[pl.ds(wid * chunk_len, chunk_len)],
                        mchunk)
        pltpu.sync_copy(tgt_hbm.at[pl.ds(row * 4 * M, 4 * M)], trow)

        def count_step(j, run_s):
            v = mchunk[pl.ds(j * _L, _L)]
            return run_s + jnp.sum((v == 1).astype(jnp.int32))

        cnt = lax.fori_loop(0, steps, count_step, jnp.int32(0))

        stage_i[...] = jnp.zeros((_L,), jnp.int32) + cnt
        pltpu.sync_copy(stage_i, shared_cnt.at[s])
        plsc.subcore_barrier()
        row_local = s // 4
        base_v = jnp.zeros((_L,), jnp.int32)
        for jj in range(3):
            pltpu.sync_copy(shared_cnt.at[row_local * 4 + jj], stage_i)
            cv = stage_i[...]
            take = (jnp.zeros((_L,), jnp.int32) + jj) < chunk
            base_v = base_v + jnp.where(take, cv, 0)

        bcopy.wait()

        def group_step(j, carry):
            run_s, acc = carry
            v = mchunk[pl.ds(j * _L, _L)]
            m = v == 1
            mi = m.astype(jnp.int32)
            g = jnp.sum(mi)

            def hot(_):
                pf = plsc.cumsum(mi)
                rank = base_v + run_s + pf - 1
                rank = jnp.clip(rank, 0, M - 1)
                boff = j * (4 * _L) + iota * 4
                tot = jnp.zeros((_L,), jnp.float32)
                for d in range(4):
                    t = plsc.load_gather(trow, [rank * 4 + d])
                    b = plsc.load_gather(bchunk, [boff + d])
                    diff = jnp.abs(t - b)
                    tot = tot + jnp.where(diff < 1.0, 0.5 * diff * diff,
                                          diff - 0.5)
                return acc + jnp.where(m, tot, 0.0)

            acc = lax.cond(g > 0, hot, lambda _: acc, 0)
            return run_s + g, acc

        _, acc = lax.fori_loop(
            0, steps, group_step,
            (jnp.int32(0), jnp.zeros((_L,), jnp.float32)))

        stage_f[...] = acc
        pltpu.sync_copy(stage_f, out_hbm.at[wid])
        stage_f[...] = jnp.zeros((_L,), jnp.float32) + cnt.astype(jnp.float32)
        pltpu.sync_copy(stage_f, out_hbm.at[_NW + wid])

    return body(match_flat, tgt_flat, bbox_flat)


def _finish_body(p_ref, o_ref):
    x = p_ref[...]
    total = jnp.sum(x[:_NW])
    n = jnp.sum(x[_NW:]) * (4.0 / _L)
    o_ref[...] = jnp.where(n > 0.0, total / n, 0.0).reshape(1, 1)


def kernel(target_bbox, rpn_match, rpn_bbox):
    B, M, _ = target_bbox.shape
    A = rpn_match.shape[1]
    match_flat = rpn_match.reshape(B * A)
    tgt_flat = target_bbox.reshape(B * M * 4)
    bbox_flat = rpn_bbox.reshape(B * A * 4)
    partials = _sc_partials(match_flat, tgt_flat, bbox_flat, A=A, M=M)
    out = pl.pallas_call(
        _finish_body,
        out_shape=jax.ShapeDtypeStruct((1, 1), jnp.float32),
    )(partials)
    return out[0, 0]
